# baseline (speedup 1.0000x reference)
import jax
import jax.numpy as jnp
from jax.experimental import pallas as pl
from jax.experimental.pallas import tpu as pltpu

_D = 768
_K = 96
_BLOCK = 8192


def _ln(x, w, b, eps=1e-5):
    mu = jnp.mean(x, axis=-1, keepdims=True)
    var = jnp.mean((x - mu) ** 2, axis=-1, keepdims=True)
    return (x - mu) / jnp.sqrt(var + eps) * w + b


def _loss_kernel(x_ref, tgt_ref, w_ref, b_ref, dic_ref, out_ref, seg_ref, cnt_ref):
    i = pl.program_id(0)
    n = pl.num_programs(0)

    @pl.when(i == 0)
    def _init():
        seg_ref[...] = jnp.zeros_like(seg_ref)
        cnt_ref[...] = jnp.zeros_like(cnt_ref)
        out_ref[...] = jnp.zeros_like(out_ref)

    x = x_ref[...]
    bsz = x.shape[0]
    mu = jnp.mean(x, axis=-1, keepdims=True)
    msq = jnp.mean(x * x, axis=-1, keepdims=True)
    r = jax.lax.rsqrt(msq - mu * mu + 1e-5)

    tgt = tgt_ref[0]
    ids = jax.lax.broadcasted_iota(jnp.int32, (_K, bsz), 0)
    onehot = ids == tgt
    oh = onehot.astype(jnp.float32)

    xs = x * r
    xh = xs.astype(jnp.bfloat16)
    xl = (xs - xh.astype(jnp.float32)).astype(jnp.bfloat16)
    seg_ref[...] += (
        jax.lax.dot(oh.astype(jnp.bfloat16), xh, preferred_element_type=jnp.float32)
        + jax.lax.dot(oh.astype(jnp.bfloat16), xl, preferred_element_type=jnp.float32))
    cnt = jnp.sum(oh, axis=1, keepdims=True)
    cnt_ref[...] += jnp.broadcast_to(cnt, cnt_ref.shape)

    @pl.when(i == n - 1)
    def _finish():
        w = w_ref[...]
        b = b_ref[...]
        dic = dic_ref[...]
        counts = cnt_ref[:, 0:1]
        ax = seg_ref[...]
        seg = w * (ax - jnp.sum(ax, axis=-1, keepdims=True) / _D) + counts * b
        s = dic + seg
        pos = jnp.sum(s * s) / _D
        llen = counts + 1.0
        rowmask = (jax.lax.broadcasted_iota(jnp.int32, (_K, 1), 0) >= 1
                   ).astype(jnp.float32)
        upd = dic + 0.1 * (s / llen) * rowmask
        upd = _ln(upd, w, b)
        usum = jnp.sum(upd * rowmask, axis=0, keepdims=True)
        neg = jnp.sum(usum * usum) / _D
        out_ref[...] = jnp.reshape(neg - pos, (1, 1))


def kernel(input_f, ln1_w, ln1_b, char_dic, target):
    flat = input_f.reshape(-1, _D)
    tokens = flat.shape[0]
    nb = tokens // _BLOCK
    tgt = target.reshape(nb, 1, _BLOCK)
    w = ln1_w.reshape(1, _D)
    b = ln1_b.reshape(1, _D)

    out = pl.pallas_call(
        _loss_kernel,
        grid=(nb,),
        in_specs=[
            pl.BlockSpec((_BLOCK, _D), lambda i: (i, 0)),
            pl.BlockSpec((1, 1, _BLOCK), lambda i: (i, 0, 0)),
            pl.BlockSpec((1, _D), lambda i: (0, 0)),
            pl.BlockSpec((1, _D), lambda i: (0, 0)),
            pl.BlockSpec((_K, _D), lambda i: (0, 0)),
        ],
        out_specs=pl.BlockSpec((1, 1), lambda i: (0, 0)),
        out_shape=jax.ShapeDtypeStruct((1, 1), jnp.float32),
        scratch_shapes=[
            pltpu.VMEM((_K, _D), jnp.float32),
            pltpu.VMEM((_K, 128), jnp.float32),
        ],
        compiler_params=pltpu.CompilerParams(
            dimension_semantics=("arbitrary",),
            vmem_limit_bytes=100 * 1024 * 1024),
    )(flat, tgt, w, b, char_dic)
    return out.reshape(1)


# speedup vs baseline: 1.0560x; 1.0560x over previous
import jax
import jax.numpy as jnp
from jax.experimental import pallas as pl
from jax.experimental.pallas import tpu as pltpu

_D = 768
_K = 96
_BLOCK = 4096


def _ln(x, w, b, eps=1e-5):
    mu = jnp.mean(x, axis=-1, keepdims=True)
    var = jnp.mean((x - mu) ** 2, axis=-1, keepdims=True)
    return (x - mu) / jnp.sqrt(var + eps) * w + b


def _loss_kernel(x_ref, tgt_ref, w_ref, b_ref, dic_ref, out_ref, seg_ref, cnt_ref):
    i = pl.program_id(0)
    n = pl.num_programs(0)

    @pl.when(i == 0)
    def _init():
        seg_ref[...] = jnp.zeros_like(seg_ref)
        cnt_ref[...] = jnp.zeros_like(cnt_ref)
        out_ref[...] = jnp.zeros_like(out_ref)

    x = x_ref[...]
    bsz = x.shape[0]
    mu = jnp.mean(x, axis=-1, keepdims=True)
    msq = jnp.mean(x * x, axis=-1, keepdims=True)
    r = jax.lax.rsqrt(msq - mu * mu + 1e-5)

    tgt = tgt_ref[0]
    ids = jax.lax.broadcasted_iota(jnp.int32, (_K, bsz), 0)
    onehot = ids == tgt
    oh = onehot.astype(jnp.float32)

    xs = x * r
    xh = xs.astype(jnp.bfloat16)
    xl = (xs - xh.astype(jnp.float32)).astype(jnp.bfloat16)
    seg_ref[...] += (
        jax.lax.dot(oh.astype(jnp.bfloat16), xh, preferred_element_type=jnp.float32)
        + jax.lax.dot(oh.astype(jnp.bfloat16), xl, preferred_element_type=jnp.float32))
    cnt = jnp.sum(oh, axis=1, keepdims=True)
    cnt_ref[...] += jnp.broadcast_to(cnt, cnt_ref.shape)

    @pl.when(i == n - 1)
    def _finish():
        w = w_ref[...]
        b = b_ref[...]
        dic = dic_ref[...]
        counts = cnt_ref[:, 0:1]
        ax = seg_ref[...]
        seg = w * (ax - jnp.sum(ax, axis=-1, keepdims=True) / _D) + counts * b
        s = dic + seg
        pos = jnp.sum(s * s) / _D
        llen = counts + 1.0
        rowmask = (jax.lax.broadcasted_iota(jnp.int32, (_K, 1), 0) >= 1
                   ).astype(jnp.float32)
        upd = dic + 0.1 * (s / llen) * rowmask
        upd = _ln(upd, w, b)
        usum = jnp.sum(upd * rowmask, axis=0, keepdims=True)
        neg = jnp.sum(usum * usum) / _D
        out_ref[...] = jnp.reshape(neg - pos, (1, 1))


def kernel(input_f, ln1_w, ln1_b, char_dic, target):
    flat = input_f.reshape(-1, _D)
    tokens = flat.shape[0]
    nb = tokens // _BLOCK
    tgt = target.reshape(nb, 1, _BLOCK)
    w = ln1_w.reshape(1, _D)
    b = ln1_b.reshape(1, _D)

    out = pl.pallas_call(
        _loss_kernel,
        grid=(nb,),
        in_specs=[
            pl.BlockSpec((_BLOCK, _D), lambda i: (i, 0)),
            pl.BlockSpec((1, 1, _BLOCK), lambda i: (i, 0, 0)),
            pl.BlockSpec((1, _D), lambda i: (0, 0)),
            pl.BlockSpec((1, _D), lambda i: (0, 0)),
            pl.BlockSpec((_K, _D), lambda i: (0, 0)),
        ],
        out_specs=pl.BlockSpec((1, 1), lambda i: (0, 0)),
        out_shape=jax.ShapeDtypeStruct((1, 1), jnp.float32),
        scratch_shapes=[
            pltpu.VMEM((_K, _D), jnp.float32),
            pltpu.VMEM((_K, 128), jnp.float32),
        ],
        compiler_params=pltpu.CompilerParams(
            dimension_semantics=("arbitrary",),
            vmem_limit_bytes=100 * 1024 * 1024),
    )(flat, tgt, w, b, char_dic)
    return out.reshape(1)
